# baseline (speedup 1.0000x reference)
import jax
import jax.numpy as jnp
from jax import lax
from jax.experimental import pallas as pl
from jax.experimental.pallas import tpu as pltpu
from jax.experimental.pallas import tpu_sc as plsc

ROWS = 1_000_000
D = 64
NIDX = 16384
G8 = 8

NC = 2
NS = 16
NW = NC * NS
RUNS_PER_TILE = NIDX // NW
SLAB = 64
NSLAB = RUNS_PER_TILE // SLAB
NSLABS_TOT = NIDX // SLAB
PB = 128
PREC = 16
NCAND = 2 * NIDX + 16


def _lane(vec, j):
    return lax.squeeze(lax.slice(vec, [j], [j + 1]), [0])


def _sc_body(sidx_hbm, rid_hbm, order_hbm, rr_hbm, rec_hbm, b_hbm, a_hbm,
             out_hbm, odbuf, recbuf, rbuf, gacc, bbuf, sxb, ridb,
             semG, semB):
    wid = lax.axis_index("s") * NC + lax.axis_index("c")

    pltpu.sync_copy(order_hbm, odbuf)
    pltpu.sync_copy(rec_hbm.at[pl.ds(wid * NSLAB * PREC, NSLAB * PREC)],
                    recbuf)

    def slab_body(sl, carry):
        rec = recbuf[pl.ds(sl * PREC, 16)]
        ps = _lane(rec, 0)
        pe = _lane(rec, 1)
        rid0 = wid * RUNS_PER_TILE + sl * SLAB
        pltpu.sync_copy(rr_hbm.at[pl.ds(rid0, SLAB)], rbuf)

        for b16 in range(SLAB // 16):
            gvec = rbuf[pl.ds(b16 * 16, 16)]
            for l in range(16):
                g = _lane(gvec, l)
                k = b16 * 16 + l
                pltpu.async_copy(
                    a_hbm.at[pl.ds(g * G8, G8)],
                    gacc.at[pl.ds(k * G8, G8)], semG)
        for k in range(SLAB):
            pltpu.make_async_copy(
                a_hbm.at[pl.ds(0, G8)], gacc.at[pl.ds(0, G8)], semG).wait()

        def batch_body(b, bc):
            bb = b * PB
            pltpu.sync_copy(sidx_hbm.at[pl.ds(bb, PB)], sxb)
            pltpu.sync_copy(rid_hbm.at[pl.ds(bb, PB)], ridb)
            pltpu.async_copy(
                b_hbm.at[odbuf.at[pl.ds(bb, PB)]], bbuf, semB).wait()
            for sub in range(PB // 16):
                rv = ridb[pl.ds(sub * 16, 16)]
                sv = sxb[pl.ds(sub * 16, 16)]
                for j in range(16):
                    pos = bb + sub * 16 + j
                    cond = jnp.logical_and(pos >= ps, pos < pe)

                    @pl.when(cond)
                    def _(sub=sub, j=j, rv=rv, sv=sv):
                        lr = _lane(rv, j) - rid0
                        r8 = _lane(sv, j) & (G8 - 1)
                        lg = lr * G8 + r8
                        for cg in range(4):
                            x = bbuf[sub * 16 + j, pl.ds(cg * 16, 16)]
                            plsc.addupdate(
                                gacc.at[lg, pl.ds(cg * 16, 16)], x)
            return bc

        lax.fori_loop(ps // PB, (pe + PB - 1) // PB, batch_body, 0)

        for b16 in range(SLAB // 16):
            gvec = rbuf[pl.ds(b16 * 16, 16)]
            for l in range(16):
                g = _lane(gvec, l)
                k = b16 * 16 + l
                pltpu.async_copy(
                    gacc.at[pl.ds(k * G8, G8)],
                    out_hbm.at[pl.ds(g * G8, G8)], semG)
        for k in range(SLAB):
            pltpu.make_async_copy(
                gacc.at[pl.ds(0, G8)], out_hbm.at[pl.ds(0, G8)],
                semG).wait()
        return carry

    lax.fori_loop(0, NSLAB, slab_body, 0)


def _sc_scatter(sidx, rid, order, run_gids, recs, B_pad, A, out_ref):
    mesh = plsc.VectorSubcoreMesh(
        core_axis_name="c", subcore_axis_name="s",
        num_cores=NC, num_subcores=NS)
    f = pl.kernel(
        _sc_body,
        out_type=(),
        mesh=mesh,
        scratch_types=[
            pltpu.VMEM((NIDX,), jnp.int32),
            pltpu.VMEM((NSLAB * PREC,), jnp.int32),
            pltpu.VMEM((SLAB,), jnp.int32),
            pltpu.VMEM((SLAB * G8, D), jnp.float32),
            pltpu.VMEM((PB, 2 * D), jnp.float32),
            pltpu.VMEM((PB,), jnp.int32),
            pltpu.VMEM((PB,), jnp.int32),
            pltpu.SemaphoreType.DMA,
            pltpu.SemaphoreType.DMA,
        ],
    )
    f(sidx, rid, order, run_gids, recs, B_pad, A, out_ref)


@jax.jit
def _scatter_add(index, A, B):
    iota = jnp.arange(NIDX, dtype=jnp.int32)
    sidx, order = lax.sort([index, iota], num_keys=1)
    sgid = sidx // G8
    is_start = jnp.concatenate(
        [jnp.ones((1,), jnp.bool_), sgid[1:] != sgid[:-1]])
    rid = jnp.cumsum(is_start.astype(jnp.int32)) - 1
    nruns = rid[NIDX - 1] + 1
    keys = jnp.where(is_start, rid, NIDX)
    run_gids = lax.sort([keys, sgid], num_keys=1)[1]
    ngrp = ROWS // G8
    d = sgid[1:] - sgid[:-1]
    gapval = jnp.max(jnp.where(d == jnp.max(d), sgid[:-1] + 1, 0))
    safe_one = jnp.where(
        sgid[0] > 0, jnp.int32(0),
        jnp.where(sgid[NIDX - 1] < ngrp - 1, jnp.int32(ngrp - 1), gapval))
    run_gids = jnp.where(iota < nruns, run_gids, safe_one)
    qid = rid // SLAB
    q = jnp.arange(NSLABS_TOT, dtype=jnp.int32)[:, None]
    ps = jnp.sum((qid[None, :] < q).astype(jnp.int32), axis=1)
    pe = jnp.sum((qid[None, :] <= q).astype(jnp.int32), axis=1)
    recs = jnp.stack([ps, pe], axis=-1)
    recs = jnp.pad(recs, ((0, 0), (0, PREC - 2))).reshape(-1)

    B_pad = jnp.pad(B, ((0, 0), (0, D)))
    ref = jax.new_ref(A)
    _sc_scatter(sidx, rid, order, run_gids, recs, B_pad, A, ref)
    return jax.freeze(ref)


def kernel(index, A, B):
    return _scatter_add(index.astype(jnp.int32), A, B)
